# baseline (speedup 1.0000x reference)
import functools
import math

import jax
import jax.numpy as jnp
from jax import lax
from jax.experimental import pallas as pl
from jax.experimental.pallas import tpu as pltpu
from jax.experimental.pallas import tpu_sc as plsc

D_MODEL = 64
SCALE = math.sqrt(D_MODEL)

NUM_CORES = 2
NUM_SUBCORES = 16
NW = NUM_CORES * NUM_SUBCORES
NBUF = 4
SB = 2 * NBUF


@functools.lru_cache(maxsize=None)
def _prescale_call(vocab):
    BLK = 2000
    assert vocab % BLK == 0

    def body(t_ref, o_ref):
        o_ref[...] = t_ref[...] * SCALE

    return pl.pallas_call(
        body,
        grid=(vocab // BLK,),
        in_specs=[pl.BlockSpec((BLK, D_MODEL), lambda i: (i, 0))],
        out_specs=pl.BlockSpec((BLK, D_MODEL), lambda i: (i, 0)),
        out_shape=jax.ShapeDtypeStruct((vocab, D_MODEL), jnp.float32),
    )


@functools.lru_cache(maxsize=None)
def _gather_call(batch, hist):
    assert batch % (NW * SB) == 0
    rows_w = batch // NW
    KB = rows_w // SB
    assert KB >= 4
    h0 = (hist // 2 + 7) & ~7
    h1 = hist - h0
    mesh = plsc.VectorSubcoreMesh(core_axis_name="c", subcore_axis_name="s")

    @functools.partial(
        pl.kernel,
        out_type=jax.ShapeDtypeStruct((batch, hist, D_MODEL), jnp.float32),
        scratch_types=[
            pltpu.VMEM((3 * SB, hist), jnp.int32),
            pltpu.VMEM((NBUF, hist, D_MODEL), jnp.float32),
            pltpu.SemaphoreType.DMA((NBUF,)),
            pltpu.SemaphoreType.DMA((NBUF,)),
            pltpu.SemaphoreType.DMA((3,)),
        ],
        mesh=mesh,
        compiler_params=pltpu.CompilerParams(use_tc_tiling_on_sc=False),
    )
    def emb(idx_hbm, table_hbm, out_hbm, idx_v, gbuf, gsem, ssem, isem):
        wid = lax.axis_index("s") * NUM_CORES + lax.axis_index("c")
        base_row = wid * rows_w

        def idx_start(k):
            slot = lax.rem(k, 3)
            pltpu.async_copy(
                idx_hbm.at[pl.ds(base_row + k * SB, SB)],
                idx_v.at[pl.ds(slot * SB, SB)],
                isem.at[slot])

        def idx_wait(k):
            slot = lax.rem(k, 3)
            pltpu.make_async_copy(
                idx_hbm.at[pl.ds(0, SB)],
                idx_v.at[pl.ds(0, SB)],
                isem.at[slot]).wait()

        def g_start(g, b):
            slot = lax.rem(lax.div(g, SB), 3)
            r = slot * SB + lax.rem(g, SB)
            pltpu.async_copy(
                table_hbm.at[idx_v.at[r, pl.ds(0, h0)]],
                gbuf.at[b, pl.ds(0, h0)],
                gsem.at[b])
            pltpu.async_copy(
                table_hbm.at[idx_v.at[r, pl.ds(h0, h1)]],
                gbuf.at[b, pl.ds(h0, h1)],
                gsem.at[b])

        def g_wait(b):
            pltpu.make_async_copy(
                table_hbm.at[pl.ds(0, hist)], gbuf.at[b], gsem.at[b]).wait()

        def s_start(g, b):
            pltpu.async_copy(
                gbuf.at[b], out_hbm.at[base_row + g], ssem.at[b])

        def s_wait(b):
            pltpu.make_async_copy(
                gbuf.at[b], out_hbm.at[0], ssem.at[b]).wait()

        def halves(K, last):
            for h in range(2):
                for b in range(NBUF):
                    g = K * SB + h * NBUF + b
                    g_wait(b)
                    s_start(g, b)
                for b in range(NBUF):
                    g = K * SB + h * NBUF + b
                    s_wait(b)
                    if not (last and h == 1):
                        g_start(g + NBUF, b)

        idx_start(0)
        idx_start(1)
        idx_wait(0)
        for b in range(NBUF):
            g_start(b, b)
        idx_wait(1)
        idx_start(2)
        halves(0, False)

        def block_body(K, c):
            idx_start(K + 2)
            idx_wait(K + 1)
            halves(K, False)
            return c
        lax.fori_loop(1, KB - 2, block_body, 0)

        idx_wait(KB - 1)
        halves(KB - 2, False)
        halves(KB - 1, True)

    return emb


def kernel(x, table):
    scaled = _prescale_call(table.shape[0])(table)
    return _gather_call(x.shape[0], x.shape[1])(x.astype(jnp.int32), scaled)


# speedup vs baseline: 1.3674x; 1.3674x over previous
import functools
import math

import jax
import jax.numpy as jnp
from jax import lax
from jax.experimental import pallas as pl
from jax.experimental.pallas import tpu as pltpu
from jax.experimental.pallas import tpu_sc as plsc

D_MODEL = 64
D_PAD = 128
SCALE = math.sqrt(D_MODEL)
LANES = 16

NUM_CORES = 2
NUM_SUBCORES = 16
NW = NUM_CORES * NUM_SUBCORES
CHUNK = 128
NBUF = 4
NCB = 2
SB = 2 * NBUF


@functools.lru_cache(maxsize=None)
def _prescale_call(vocab):
    BLK = 2000
    assert vocab % BLK == 0

    def body(t_ref, o_ref):
        o_ref[:, 0:D_MODEL] = t_ref[...] * SCALE
        o_ref[:, D_MODEL:D_PAD] = jnp.zeros((BLK, D_MODEL), jnp.float32)

    return pl.pallas_call(
        body,
        grid=(vocab // BLK,),
        in_specs=[pl.BlockSpec((BLK, D_MODEL), lambda i: (i, 0))],
        out_specs=pl.BlockSpec((BLK, D_PAD), lambda i: (i, 0)),
        out_shape=jax.ShapeDtypeStruct((vocab, D_PAD), jnp.float32),
    )


@functools.lru_cache(maxsize=None)
def _gather_call(B):
    assert B % (NW * CHUNK * SB) == 0
    n_chunks_w = B // (NW * CHUNK)
    KB = n_chunks_w // SB
    assert KB >= 4
    mesh = plsc.VectorSubcoreMesh(core_axis_name="c", subcore_axis_name="s")

    @functools.partial(
        pl.kernel,
        out_type=jax.ShapeDtypeStruct((B, D_MODEL), jnp.float32),
        scratch_types=[
            pltpu.VMEM((3, SB, CHUNK), jnp.int32),
            pltpu.VMEM((NBUF, CHUNK, D_PAD), jnp.float32),
            pltpu.VMEM((NCB, CHUNK, D_MODEL), jnp.float32),
            pltpu.SemaphoreType.DMA((NBUF,)),
            pltpu.SemaphoreType.DMA((NCB,)),
            pltpu.SemaphoreType.DMA((3,)),
        ],
        mesh=mesh,
        compiler_params=pltpu.CompilerParams(use_tc_tiling_on_sc=True),
    )
    def emb(idx_hbm, table_hbm, out_hbm, idx_v, gbuf, cbuf, gsem, ssem, isem):
        wid = lax.axis_index("s") * NUM_CORES + lax.axis_index("c")
        base_sb = wid * KB
        base_chunk = base_sb * SB

        def idx_start(k):
            slot = lax.rem(k, 3)
            pltpu.async_copy(
                idx_hbm.at[pl.ds(base_sb + k, 1)],
                idx_v.at[pl.ds(slot, 1)],
                isem.at[slot])

        def idx_wait(k):
            slot = lax.rem(k, 3)
            pltpu.make_async_copy(
                idx_hbm.at[pl.ds(0, 1)],
                idx_v.at[pl.ds(0, 1)],
                isem.at[slot]).wait()

        def g_start(g, b):
            slot = lax.rem(lax.div(g, SB), 3)
            row = lax.rem(g, SB)
            pltpu.async_copy(
                table_hbm.at[idx_v.at[slot, row]],
                gbuf.at[b],
                gsem.at[b])

        def g_wait(b):
            pltpu.make_async_copy(
                table_hbm.at[pl.ds(0, CHUNK)], gbuf.at[b], gsem.at[b]).wait()

        def compact(b, c):
            def row_body(i, carry):
                for h in range(D_MODEL // LANES):
                    s = pl.ds(h * LANES, LANES)
                    cbuf[c, i, s] = gbuf[b, i, s]
                return carry
            lax.fori_loop(0, CHUNK, row_body, 0, unroll=4)

        def s_start(g, c):
            row0 = (base_chunk + g) * CHUNK
            pltpu.async_copy(
                cbuf.at[c],
                out_hbm.at[pl.ds(row0, CHUNK)],
                ssem.at[c])

        def s_wait(c):
            pltpu.make_async_copy(
                cbuf.at[c],
                out_hbm.at[pl.ds(0, CHUNK)],
                ssem.at[c]).wait()

        def sblock(K, first, last):
            for p in range(SB):
                g = K * SB + p
                b = p % NBUF
                c = p % NCB
                g_wait(b)
                if not (first and p < NCB):
                    s_wait(c)
                compact(b, c)
                s_start(g, c)
                if not (last and p >= SB - NBUF):
                    g_start(g + NBUF, b)

        idx_start(0)
        idx_start(1)
        idx_wait(0)
        for b in range(NBUF):
            g_start(b, b)
        idx_wait(1)
        idx_start(2)
        sblock(0, True, False)

        def block_body(K, carry):
            idx_start(K + 2)
            idx_wait(K + 1)
            sblock(K, False, False)
            return carry
        lax.fori_loop(1, KB - 2, block_body, 0)

        idx_wait(KB - 1)
        sblock(KB - 2, False, False)
        sblock(KB - 1, False, True)
        for c in range(NCB):
            s_wait(c)

    return emb


def kernel(x, table):
    B = x.shape[0] * x.shape[1]
    idx3d = x.reshape((B // (SB * CHUNK), SB, CHUNK)).astype(jnp.int32)
    scaled = _prescale_call(table.shape[0])(table)
    out = _gather_call(B)(idx3d, scaled)
    return out.reshape(x.shape[0], x.shape[1], D_MODEL)
